# baseline (speedup 1.0000x reference)
import functools

import jax
import jax.numpy as jnp
from jax import lax
from jax.experimental import pallas as pl
from jax.experimental.pallas import tpu as pltpu
from jax.experimental.pallas import tpu_sc as plsc

N_TOKENS = 8192
NUM_CLASSES = 1024
EMBED_DIM = 128

NC = 2
NS = 16
NW = NC * NS
TOK_PER_W = N_TOKENS // NW
CHUNK = 128
CHUNKS_PER_W = TOK_PER_W // CHUNK
ROWS_PER_S = NUM_CLASSES // NS
CNT_W = 128

B_BLK = 512
B_STEPS = N_TOKENS // B_BLK

_PREC = jax.lax.Precision.DEFAULT


def _sc_segsum_kernel(x_hbm, y_hbm, ones_hbm, zsum_hbm, zcnt_hbm,
                      psum_hbm, pcnt_hbm,
                      x_v, idx_v, ones_v, acc_sum, acc_cnt):
    c = lax.axis_index("c")
    s = lax.axis_index("s")
    wid = s * NC + c

    pltpu.sync_copy(zsum_hbm.at[pl.ds(s * ROWS_PER_S, ROWS_PER_S)],
                    acc_sum.at[pl.ds(s * ROWS_PER_S, ROWS_PER_S)])
    pltpu.sync_copy(zcnt_hbm.at[pl.ds(s * ROWS_PER_S, ROWS_PER_S)],
                    acc_cnt.at[pl.ds(s * ROWS_PER_S, ROWS_PER_S)])
    pltpu.sync_copy(x_hbm.at[pl.ds(wid * TOK_PER_W, TOK_PER_W)], x_v)
    pltpu.sync_copy(y_hbm.at[pl.ds(wid * CHUNKS_PER_W, CHUNKS_PER_W)], idx_v)
    pltpu.sync_copy(ones_hbm, ones_v)
    plsc.subcore_barrier()

    for j in range(CHUNKS_PER_W):
        pltpu.sync_copy(x_v.at[pl.ds(j * CHUNK, CHUNK)],
                        acc_sum.at[idx_v.at[j]], add=True)
        pltpu.sync_copy(ones_v, acc_cnt.at[idx_v.at[j]], add=True)
    plsc.subcore_barrier()

    pltpu.sync_copy(acc_sum.at[pl.ds(s * ROWS_PER_S, ROWS_PER_S)],
                    psum_hbm.at[c, pl.ds(s * ROWS_PER_S, ROWS_PER_S)])
    pltpu.sync_copy(acc_cnt.at[pl.ds(s * ROWS_PER_S, ROWS_PER_S)],
                    pcnt_hbm.at[c, pl.ds(s * ROWS_PER_S, ROWS_PER_S)])


def _sc_segsum(x, y_true):
    y2 = y_true.reshape(NW * CHUNKS_PER_W, CHUNK)
    ones = jnp.ones((CHUNK, CNT_W), jnp.float32)
    zsum = jnp.zeros((NUM_CLASSES, EMBED_DIM), jnp.float32)
    zcnt = jnp.zeros((NUM_CLASSES, CNT_W), jnp.float32)
    mesh = plsc.VectorSubcoreMesh(core_axis_name="c", subcore_axis_name="s")
    fn = pl.kernel(
        _sc_segsum_kernel,
        out_type=(
            jax.ShapeDtypeStruct((NC, NUM_CLASSES, EMBED_DIM), jnp.float32),
            jax.ShapeDtypeStruct((NC, NUM_CLASSES, CNT_W), jnp.float32),
        ),
        mesh=mesh,
        scratch_types=[
            pltpu.VMEM((TOK_PER_W, EMBED_DIM), jnp.float32),
            pltpu.VMEM((CHUNKS_PER_W, CHUNK), jnp.int32),
            pltpu.VMEM((CHUNK, CNT_W), jnp.float32),
            pltpu.VMEM_SHARED((NUM_CLASSES, EMBED_DIM), jnp.float32),
            pltpu.VMEM_SHARED((NUM_CLASSES, CNT_W), jnp.float32),
        ],
    )
    return fn(x, y2, ones, zsum, zcnt)


def _update_kernel(psum_ref, pcnt_ref, p_ref, c_ref, u_ref, usq_ref):
    sums = psum_ref[0] + psum_ref[1]
    cnt8 = pcnt_ref[0] + pcnt_ref[1]
    cnt = cnt8[:, 0:1]
    new = sums / jnp.maximum(cnt, 1.0)
    c = c_ref[...]
    u = jnp.where(cnt > 0.0, (c * p_ref[...] + new) / (c + 1.0), p_ref[...])
    u_ref[...] = u
    usq_ref[...] = jax.lax.dot_general(
        jnp.ones((1, EMBED_DIM), jnp.float32), u * u,
        (((1,), (1,)), ((), ())),
        precision=_PREC, preferred_element_type=jnp.float32)


def _dist_kernel(x_ref, u_ref, usq_ref, o_ref):
    x = x_ref[...]
    d = jax.lax.dot_general(x, u_ref[...], (((1,), (1,)), ((), ())),
                            precision=_PREC, preferred_element_type=jnp.float32)
    xsq = jax.lax.dot_general(x * x, jnp.ones((1, EMBED_DIM), jnp.float32),
                              (((1,), (1,)), ((), ())),
                              precision=_PREC, preferred_element_type=jnp.float32)
    o_ref[...] = -jnp.maximum(xsq + usq_ref[...] - 2.0 * d, 0.0)


def kernel(x, y_true, prototypes, counter):
    c2 = counter.reshape(NUM_CLASSES, 1)
    psum, pcnt = _sc_segsum(x, y_true)

    u, usq = pl.pallas_call(
        _update_kernel,
        in_specs=[
            pl.BlockSpec((NC, NUM_CLASSES, EMBED_DIM), lambda: (0, 0, 0)),
            pl.BlockSpec((NC, NUM_CLASSES, CNT_W), lambda: (0, 0, 0)),
            pl.BlockSpec((NUM_CLASSES, EMBED_DIM), lambda: (0, 0)),
            pl.BlockSpec((NUM_CLASSES, 1), lambda: (0, 0)),
        ],
        out_specs=[
            pl.BlockSpec((NUM_CLASSES, EMBED_DIM), lambda: (0, 0)),
            pl.BlockSpec((1, NUM_CLASSES), lambda: (0, 0)),
        ],
        out_shape=[
            jax.ShapeDtypeStruct((NUM_CLASSES, EMBED_DIM), jnp.float32),
            jax.ShapeDtypeStruct((1, NUM_CLASSES), jnp.float32),
        ],
    )(psum, pcnt, prototypes, c2)

    out = pl.pallas_call(
        _dist_kernel,
        grid=(B_STEPS,),
        in_specs=[
            pl.BlockSpec((B_BLK, EMBED_DIM), lambda i: (i, 0)),
            pl.BlockSpec((NUM_CLASSES, EMBED_DIM), lambda i: (0, 0)),
            pl.BlockSpec((1, NUM_CLASSES), lambda i: (0, 0)),
        ],
        out_specs=pl.BlockSpec((B_BLK, NUM_CLASSES), lambda i: (i, 0)),
        out_shape=jax.ShapeDtypeStruct((N_TOKENS, NUM_CLASSES), jnp.float32),
    )(x, u, usq)
    return out


# speedup vs baseline: 1.0520x; 1.0520x over previous
import jax
import jax.numpy as jnp
from jax import lax
from jax.experimental import pallas as pl
from jax.experimental.pallas import tpu as pltpu
from jax.experimental.pallas import tpu_sc as plsc

N_TOKENS = 8192
NUM_CLASSES = 1024
EMBED_DIM = 128

NC = 2
NS = 16
NW = NC * NS
TOK_PER_W = N_TOKENS // NW
CHUNK = 128
CHUNKS_PER_W = TOK_PER_W // CHUNK
ROWS_PER_S = NUM_CLASSES // NS
CNT_W = 128

B_BLK = 512
B_STEPS = N_TOKENS // B_BLK

_PREC = jax.lax.Precision.DEFAULT


def _sc_segsum_kernel(x_hbm, y_hbm, zero_hbm,
                      psum_hbm, pcnt_hbm,
                      x_v, idx_v, ones_v, acc_sum, acc_cnt,
                      sem_x, sem_y):
    c = lax.axis_index("c")
    s = lax.axis_index("s")
    wid = s * NC + c

    cp_x = pltpu.async_copy(x_hbm.at[pl.ds(wid * TOK_PER_W, TOK_PER_W)],
                            x_v, sem_x)
    cp_y = pltpu.async_copy(y_hbm.at[pl.ds(wid * CHUNKS_PER_W, CHUNKS_PER_W)],
                            idx_v, sem_y)

    def _fill_ones(r, carry):
        for k in range(CHUNK // 16):
            ones_v[r, pl.ds(k * 16, 16)] = jnp.ones((16,), jnp.float32)
        return carry
    lax.fori_loop(0, CHUNK, _fill_ones, 0)

    pltpu.sync_copy(zero_hbm, acc_sum.at[pl.ds(s * ROWS_PER_S, ROWS_PER_S)])
    pltpu.sync_copy(zero_hbm, acc_cnt.at[pl.ds(s * ROWS_PER_S, ROWS_PER_S)])

    cp_x.wait()
    cp_y.wait()
    plsc.subcore_barrier()

    for j in range(CHUNKS_PER_W):
        pltpu.sync_copy(x_v.at[pl.ds(j * CHUNK, CHUNK)],
                        acc_sum.at[idx_v.at[j]], add=True)
        pltpu.sync_copy(ones_v, acc_cnt.at[idx_v.at[j]], add=True)
    plsc.subcore_barrier()

    pltpu.sync_copy(acc_sum.at[pl.ds(s * ROWS_PER_S, ROWS_PER_S)],
                    psum_hbm.at[c, pl.ds(s * ROWS_PER_S, ROWS_PER_S)])
    pltpu.sync_copy(acc_cnt.at[pl.ds(s * ROWS_PER_S, ROWS_PER_S), pl.ds(0, CNT_W)],
                    pcnt_hbm.at[c, pl.ds(s * ROWS_PER_S, ROWS_PER_S)])


def _sc_segsum(x, y_true):
    y2 = y_true.reshape(NW * CHUNKS_PER_W, CHUNK)
    zero = jnp.zeros((ROWS_PER_S, EMBED_DIM), jnp.float32)
    mesh = plsc.VectorSubcoreMesh(core_axis_name="c", subcore_axis_name="s")
    fn = pl.kernel(
        _sc_segsum_kernel,
        out_type=(
            jax.ShapeDtypeStruct((NC, NUM_CLASSES, EMBED_DIM), jnp.float32),
            jax.ShapeDtypeStruct((NC, NUM_CLASSES, CNT_W), jnp.float32),
        ),
        mesh=mesh,
        scratch_types=[
            pltpu.VMEM((TOK_PER_W, EMBED_DIM), jnp.float32),
            pltpu.VMEM((CHUNKS_PER_W, CHUNK), jnp.int32),
            pltpu.VMEM((CHUNK, EMBED_DIM), jnp.float32),
            pltpu.VMEM_SHARED((NUM_CLASSES, EMBED_DIM), jnp.float32),
            pltpu.VMEM_SHARED((NUM_CLASSES, EMBED_DIM), jnp.float32),
            pltpu.SemaphoreType.DMA,
            pltpu.SemaphoreType.DMA,
        ],
    )
    return fn(x, y2, zero)


def _dist_kernel(psum_ref, pcnt_ref, p_ref, c_ref, x_ref, o_ref,
                 u2_ref, usq_ref):
    i = pl.program_id(0)

    @pl.when(i == 0)
    def _update():
        sums = psum_ref[0] + psum_ref[1]
        cnt = pcnt_ref[0, :, 0:1] + pcnt_ref[1, :, 0:1]
        new = sums / jnp.maximum(cnt, 1.0)
        c = c_ref[...]
        u = jnp.where(cnt > 0.0, (c * p_ref[...] + new) / (c + 1.0),
                      p_ref[...])
        u2_ref[...] = u + u
        usq_ref[...] = jax.lax.dot_general(
            jnp.ones((1, EMBED_DIM), jnp.float32), u * u,
            (((1,), (1,)), ((), ())),
            precision=_PREC, preferred_element_type=jnp.float32)

    x = x_ref[...]
    d2 = jax.lax.dot_general(x, u2_ref[...], (((1,), (1,)), ((), ())),
                             precision=_PREC,
                             preferred_element_type=jnp.float32)
    xsq = jax.lax.dot_general(x * x, jnp.ones((1, EMBED_DIM), jnp.float32),
                              (((1,), (1,)), ((), ())),
                              precision=_PREC,
                              preferred_element_type=jnp.float32)
    o_ref[...] = jnp.minimum(d2 - xsq - usq_ref[...], 0.0)


def kernel(x, y_true, prototypes, counter):
    c2 = counter.reshape(NUM_CLASSES, 1)
    psum, pcnt = _sc_segsum(x, y_true)

    out = pl.pallas_call(
        _dist_kernel,
        grid=(B_STEPS,),
        in_specs=[
            pl.BlockSpec((NC, NUM_CLASSES, EMBED_DIM), lambda i: (0, 0, 0)),
            pl.BlockSpec((NC, NUM_CLASSES, CNT_W), lambda i: (0, 0, 0)),
            pl.BlockSpec((NUM_CLASSES, EMBED_DIM), lambda i: (0, 0)),
            pl.BlockSpec((NUM_CLASSES, 1), lambda i: (0, 0)),
            pl.BlockSpec((B_BLK, EMBED_DIM), lambda i: (i, 0)),
        ],
        out_specs=pl.BlockSpec((B_BLK, NUM_CLASSES), lambda i: (i, 0)),
        out_shape=jax.ShapeDtypeStruct((N_TOKENS, NUM_CLASSES), jnp.float32),
        scratch_shapes=[
            pltpu.VMEM((NUM_CLASSES, EMBED_DIM), jnp.float32),
            pltpu.VMEM((1, NUM_CLASSES), jnp.float32),
        ],
    )(psum, pcnt, prototypes, c2, x)
    return out


# speedup vs baseline: 1.2651x; 1.2026x over previous
import jax
import jax.numpy as jnp
from jax.experimental import pallas as pl
from jax.experimental.pallas import tpu as pltpu

N_TOKENS = 8192
NUM_CLASSES = 1024
EMBED_DIM = 128

BLK = 512
STEPS = N_TOKENS // BLK

_PREC = jax.lax.Precision.DEFAULT


def _fused_kernel(x_ref, y_ref, p_ref, c_ref, o_ref, sums_ref, cnt_ref,
                  u2_ref, usq_ref):
    p = pl.program_id(0)
    i = pl.program_id(1)

    @pl.when((p == 0) & (i == 0))
    def _init():
        sums_ref[...] = jnp.zeros_like(sums_ref)
        cnt_ref[...] = jnp.zeros_like(cnt_ref)

    @pl.when(p == 0)
    def _accum():
        y_blk = y_ref[...]
        cls = jax.lax.broadcasted_iota(jnp.int32, (BLK, NUM_CLASSES), 1)
        oh = (y_blk == cls).astype(jnp.float32)
        sums_ref[...] += jax.lax.dot_general(
            oh, x_ref[...], (((0,), (0,)), ((), ())),
            precision=_PREC, preferred_element_type=jnp.float32)
        cnt_ref[...] += jax.lax.dot_general(
            oh, jnp.ones((BLK, 8), jnp.float32), (((0,), (0,)), ((), ())),
            precision=_PREC, preferred_element_type=jnp.float32)

    @pl.when((p == 0) & (i == STEPS - 1))
    def _update():
        cnt = cnt_ref[:, 0:1]
        new = sums_ref[...] / jnp.maximum(cnt, 1.0)
        c = c_ref[...]
        u = jnp.where(cnt > 0.0, (c * p_ref[...] + new) / (c + 1.0),
                      p_ref[...])
        u2_ref[...] = u + u
        usq_ref[...] = jax.lax.dot_general(
            jnp.ones((1, EMBED_DIM), jnp.float32), u * u,
            (((1,), (1,)), ((), ())),
            precision=_PREC, preferred_element_type=jnp.float32)

    @pl.when(p == 1)
    def _dist():
        x = x_ref[...]
        d2 = jax.lax.dot_general(x, u2_ref[...], (((1,), (1,)), ((), ())),
                                 precision=_PREC,
                                 preferred_element_type=jnp.float32)
        xsq = jax.lax.dot_general(x * x, jnp.ones((1, EMBED_DIM), jnp.float32),
                                  (((1,), (1,)), ((), ())),
                                  precision=_PREC,
                                  preferred_element_type=jnp.float32)
        o_ref[...] = jnp.minimum(d2 - xsq - usq_ref[...], 0.0)


def kernel(x, y_true, prototypes, counter):
    y2 = y_true.reshape(N_TOKENS, 1)
    c2 = counter.reshape(NUM_CLASSES, 1)
    out = pl.pallas_call(
        _fused_kernel,
        grid=(2, STEPS),
        in_specs=[
            pl.BlockSpec((BLK, EMBED_DIM), lambda p, i: (i, 0)),
            pl.BlockSpec((BLK, 1), lambda p, i: (i, 0)),
            pl.BlockSpec((NUM_CLASSES, EMBED_DIM), lambda p, i: (0, 0)),
            pl.BlockSpec((NUM_CLASSES, 1), lambda p, i: (0, 0)),
        ],
        out_specs=pl.BlockSpec((BLK, NUM_CLASSES), lambda p, i: (i * p, 0)),
        out_shape=jax.ShapeDtypeStruct((N_TOKENS, NUM_CLASSES), jnp.float32),
        scratch_shapes=[
            pltpu.VMEM((NUM_CLASSES, EMBED_DIM), jnp.float32),
            pltpu.VMEM((NUM_CLASSES, 8), jnp.float32),
            pltpu.VMEM((NUM_CLASSES, EMBED_DIM), jnp.float32),
            pltpu.VMEM((1, NUM_CLASSES), jnp.float32),
        ],
    )(x, y2, prototypes, c2)
    return out


# speedup vs baseline: 1.5275x; 1.2074x over previous
import jax
import jax.numpy as jnp
from jax.experimental import pallas as pl
from jax.experimental.pallas import tpu as pltpu

N_TOKENS = 8192
NUM_CLASSES = 1024
EMBED_DIM = 128

BLK = 1024
STEPS = N_TOKENS // BLK

_PREC = jax.lax.Precision.DEFAULT


def _fused_kernel(x_ref, y_ref, p_ref, c_ref, o_ref, sums_ref, cnt_ref,
                  u2_ref, usq_ref):
    p = pl.program_id(0)
    i = pl.program_id(1)

    @pl.when((p == 0) & (i == 0))
    def _init():
        sums_ref[...] = jnp.zeros_like(sums_ref)
        cnt_ref[...] = jnp.zeros_like(cnt_ref)

    @pl.when(p == 0)
    def _accum():
        y_blk = y_ref[...]
        cls = jax.lax.broadcasted_iota(jnp.int32, (BLK, NUM_CLASSES), 1)
        oh = (y_blk == cls).astype(jnp.float32)
        sums_ref[...] += jax.lax.dot_general(
            oh, x_ref[...], (((0,), (0,)), ((), ())),
            precision=_PREC, preferred_element_type=jnp.float32)
        cnt_ref[...] += jax.lax.dot_general(
            oh, jnp.ones((BLK, 8), jnp.float32), (((0,), (0,)), ((), ())),
            precision=_PREC, preferred_element_type=jnp.float32)

    @pl.when((p == 0) & (i == STEPS - 1))
    def _update():
        cnt = cnt_ref[:, 0:1]
        new = sums_ref[...] / jnp.maximum(cnt, 1.0)
        c = c_ref[...]
        u = jnp.where(cnt > 0.0, (c * p_ref[...] + new) / (c + 1.0),
                      p_ref[...])
        u2_ref[...] = u + u
        usq_ref[...] = jax.lax.dot_general(
            jnp.ones((1, EMBED_DIM), jnp.float32), u * u,
            (((1,), (1,)), ((), ())),
            precision=_PREC, preferred_element_type=jnp.float32)

    @pl.when(p == 1)
    def _dist():
        x = x_ref[...]
        d2 = jax.lax.dot_general(x, u2_ref[...], (((1,), (1,)), ((), ())),
                                 precision=_PREC,
                                 preferred_element_type=jnp.float32)
        xsq = jax.lax.dot_general(x * x, jnp.ones((1, EMBED_DIM), jnp.float32),
                                  (((1,), (1,)), ((), ())),
                                  precision=_PREC,
                                  preferred_element_type=jnp.float32)
        o_ref[...] = jnp.minimum(d2 - xsq - usq_ref[...], 0.0)


def kernel(x, y_true, prototypes, counter):
    y2 = y_true.reshape(N_TOKENS, 1)
    c2 = counter.reshape(NUM_CLASSES, 1)
    out = pl.pallas_call(
        _fused_kernel,
        grid=(2, STEPS),
        in_specs=[
            pl.BlockSpec((BLK, EMBED_DIM), lambda p, i: (i, 0)),
            pl.BlockSpec((BLK, 1), lambda p, i: (i, 0)),
            pl.BlockSpec((NUM_CLASSES, EMBED_DIM), lambda p, i: (0, 0)),
            pl.BlockSpec((NUM_CLASSES, 1), lambda p, i: (0, 0)),
        ],
        out_specs=pl.BlockSpec((BLK, NUM_CLASSES), lambda p, i: (i * p, 0)),
        out_shape=jax.ShapeDtypeStruct((N_TOKENS, NUM_CLASSES), jnp.float32),
        scratch_shapes=[
            pltpu.VMEM((NUM_CLASSES, EMBED_DIM), jnp.float32),
            pltpu.VMEM((NUM_CLASSES, 8), jnp.float32),
            pltpu.VMEM((NUM_CLASSES, EMBED_DIM), jnp.float32),
            pltpu.VMEM((1, NUM_CLASSES), jnp.float32),
        ],
    )(x, y2, prototypes, c2)
    return out


# speedup vs baseline: 1.6370x; 1.0717x over previous
import jax
import jax.numpy as jnp
from jax.experimental import pallas as pl
from jax.experimental.pallas import tpu as pltpu

N_TOKENS = 8192
NUM_CLASSES = 1024
EMBED_DIM = 128

BLK = 2048
STEPS = N_TOKENS // BLK

_PREC = jax.lax.Precision.DEFAULT


def _fused_kernel(x_ref, y_ref, p_ref, c_ref, o_ref, sums_ref, cnt_ref,
                  u2_ref, usq_ref):
    p = pl.program_id(0)
    i = pl.program_id(1)

    @pl.when((p == 0) & (i == 0))
    def _init():
        sums_ref[...] = jnp.zeros_like(sums_ref)
        cnt_ref[...] = jnp.zeros_like(cnt_ref)

    @pl.when(p == 0)
    def _accum():
        y_blk = y_ref[...]
        cls = jax.lax.broadcasted_iota(jnp.int32, (BLK, NUM_CLASSES), 1)
        oh = (y_blk == cls).astype(jnp.float32)
        sums_ref[...] += jax.lax.dot_general(
            oh, x_ref[...], (((0,), (0,)), ((), ())),
            precision=_PREC, preferred_element_type=jnp.float32)
        cnt_ref[...] += jax.lax.dot_general(
            oh, jnp.ones((BLK, 8), jnp.float32), (((0,), (0,)), ((), ())),
            precision=_PREC, preferred_element_type=jnp.float32)

    @pl.when((p == 0) & (i == STEPS - 1))
    def _update():
        cnt = cnt_ref[:, 0:1]
        new = sums_ref[...] / jnp.maximum(cnt, 1.0)
        c = c_ref[...]
        u = jnp.where(cnt > 0.0, (c * p_ref[...] + new) / (c + 1.0),
                      p_ref[...])
        u2_ref[...] = u + u
        usq_ref[...] = jax.lax.dot_general(
            jnp.ones((1, EMBED_DIM), jnp.float32), u * u,
            (((1,), (1,)), ((), ())),
            precision=_PREC, preferred_element_type=jnp.float32)

    @pl.when(p == 1)
    def _dist():
        x = x_ref[...]
        d2 = jax.lax.dot_general(x, u2_ref[...], (((1,), (1,)), ((), ())),
                                 precision=_PREC,
                                 preferred_element_type=jnp.float32)
        xsq = jax.lax.dot_general(x * x, jnp.ones((1, EMBED_DIM), jnp.float32),
                                  (((1,), (1,)), ((), ())),
                                  precision=_PREC,
                                  preferred_element_type=jnp.float32)
        o_ref[...] = jnp.minimum(d2 - xsq - usq_ref[...], 0.0)


def kernel(x, y_true, prototypes, counter):
    y2 = y_true.reshape(N_TOKENS, 1)
    c2 = counter.reshape(NUM_CLASSES, 1)
    out = pl.pallas_call(
        _fused_kernel,
        grid=(2, STEPS),
        in_specs=[
            pl.BlockSpec((BLK, EMBED_DIM), lambda p, i: (i, 0)),
            pl.BlockSpec((BLK, 1), lambda p, i: (i, 0)),
            pl.BlockSpec((NUM_CLASSES, EMBED_DIM), lambda p, i: (0, 0)),
            pl.BlockSpec((NUM_CLASSES, 1), lambda p, i: (0, 0)),
        ],
        out_specs=pl.BlockSpec((BLK, NUM_CLASSES), lambda p, i: (i * p, 0)),
        out_shape=jax.ShapeDtypeStruct((N_TOKENS, NUM_CLASSES), jnp.float32),
        scratch_shapes=[
            pltpu.VMEM((NUM_CLASSES, EMBED_DIM), jnp.float32),
            pltpu.VMEM((NUM_CLASSES, 8), jnp.float32),
            pltpu.VMEM((NUM_CLASSES, EMBED_DIM), jnp.float32),
            pltpu.VMEM((1, NUM_CLASSES), jnp.float32),
        ],
    )(x, y2, prototypes, c2)
    return out
